# baseline (speedup 1.0000x reference)
import functools

import jax
import jax.numpy as jnp
from jax import lax
from jax.experimental import pallas as pl
from jax.experimental.pallas import tpu as pltpu
from jax.experimental.pallas import tpu_sc as plsc

B, T = 1024, 50
NUM_FIELDS = 26
CONT_SIZE = 13
INPUT_SIZE = NUM_FIELDS + CONT_SIZE
VOCAB = 100000
D_MODEL = 32

N = B * T
NC, NS, LANES = 2, 16, 16
NW = NC * NS
TPW = N // NW

VCHUNK = 2048
NVC = 49
VP = NVC * VCHUNK
QP = VP // 4
PROWS = NUM_FIELDS * QP
PD = 128

CT = 16
NCHUNK = TPW // CT
RPC = CT * NUM_FIELDS
GSUB = 32
NSUB = RPC // GSUB


def _tree_sum(vals):
    while len(vals) > 1:
        nxt = [vals[k] + vals[k + 1] for k in range(0, len(vals) - 1, 2)]
        if len(vals) % 2:
            nxt.append(vals[-1])
        vals = nxt
    return vals[0]


def _proj_body(tabT_ref, w_ref, out_ref):
    blk = tabT_ref[0]
    y = lax.dot_general(
        blk, w_ref[...], (((0,), (1,)), ((), ())),
        preferred_element_type=jnp.float32,
    )
    groups = []
    for g in range(VCHUNK // 512):
        base = g * 512
        groups.append(
            jnp.concatenate(
                [y[base + s * 128:base + (s + 1) * 128, :] for s in range(4)],
                axis=1,
            )
        )
    out_ref[...] = jnp.concatenate(groups, axis=0)


def _tc_project(tabT, We):
    return pl.pallas_call(
        _proj_body,
        grid=(NUM_FIELDS, NVC),
        in_specs=[
            pl.BlockSpec((1, D_MODEL, VCHUNK), lambda i, c: (i, 0, c)),
            pl.BlockSpec((D_MODEL, D_MODEL), lambda i, c: (0, 0)),
        ],
        out_specs=pl.BlockSpec(
            (VCHUNK // 4, PD), lambda i, c: (i * NVC + c, 0)
        ),
        out_shape=jax.ShapeDtypeStruct((PROWS, PD), jnp.float32),
    )(tabT, We)


def _sc_embed_body(x_hbm, tab_hbm, emb_hbm, xv, rid, slotb, rows, emb_v, sem):
    wid = lax.axis_index("s") * NC + lax.axis_index("c")
    base0 = wid * TPW
    iota = lax.iota(jnp.int32, LANES)

    @pl.loop(0, NCHUNK)
    def _chunk(g):
        base = base0 + g * CT
        pltpu.sync_copy(x_hbm.at[pl.ds(base * INPUT_SIZE, CT * INPUT_SIZE)], xv)

        for i in range(NUM_FIELDS):
            r = plsc.load_gather(xv, [iota * INPUT_SIZE + i])
            q = ((r >> 9) << 7) + (r & 127) + i * QP
            f = i * CT
            rid[f // GSUB, pl.ds(f % GSUB, LANES)] = q
            slotb[pl.ds(i * LANES, LANES)] = ((r >> 7) & 3) << 5

        descs = [
            pltpu.async_copy(
                tab_hbm.at[rid.at[j]],
                rows.at[pl.ds(j * GSUB, GSUB)],
                sem,
            )
            for j in range(NSUB)
        ]
        for d in descs:
            d.wait()

        cols = [slotb[pl.ds(i * LANES, LANES)] for i in range(NUM_FIELDS)]
        rowsv = [iota + i * CT for i in range(NUM_FIELDS)]
        for d in range(D_MODEL):
            vals = [
                plsc.load_gather(rows, [rowsv[i], cols[i] + d])
                for i in range(NUM_FIELDS)
            ]
            plsc.store_scatter(
                emb_v, [iota, jnp.full((LANES,), d, jnp.int32)], _tree_sum(vals)
            )

        pltpu.sync_copy(emb_v, emb_hbm.at[pl.ds(base, CT)])


_sc_embed = functools.partial(
    pl.kernel,
    out_type=jax.ShapeDtypeStruct((N, D_MODEL), jnp.float32),
    mesh=plsc.VectorSubcoreMesh(
        core_axis_name="c", subcore_axis_name="s", num_cores=NC, num_subcores=NS
    ),
    compiler_params=pltpu.CompilerParams(
        needs_layout_passes=False, use_tc_tiling_on_sc=True
    ),
    scratch_types=[
        pltpu.VMEM((CT * INPUT_SIZE,), jnp.int32),
        pltpu.VMEM((NSUB, GSUB), jnp.int32),
        pltpu.VMEM((NUM_FIELDS * LANES,), jnp.int32),
        pltpu.VMEM((RPC, PD), jnp.float32),
        pltpu.VMEM((CT, D_MODEL), jnp.float32),
        pltpu.SemaphoreType.DMA,
    ],
)(_sc_embed_body)


ROWS_BLK = 2048


def _tc_cont_body(s_ref, x_ref, w_ref, b_ref, out_ref):
    cont = x_ref[:, NUM_FIELDS:].astype(jnp.float32)
    out = lax.dot_general(
        cont, w_ref[:, D_MODEL:], (((1,), (1,)), ((), ())),
        preferred_element_type=jnp.float32,
    )
    out_ref[...] = out + s_ref[...] + b_ref[...]


def _tc_cont(s, x2d, W, b2d):
    return pl.pallas_call(
        _tc_cont_body,
        grid=(N // ROWS_BLK,),
        in_specs=[
            pl.BlockSpec((ROWS_BLK, D_MODEL), lambda i: (i, 0)),
            pl.BlockSpec((ROWS_BLK, INPUT_SIZE), lambda i: (i, 0)),
            pl.BlockSpec((D_MODEL, D_MODEL + CONT_SIZE), lambda i: (0, 0)),
            pl.BlockSpec((1, D_MODEL), lambda i: (0, 0)),
        ],
        out_specs=pl.BlockSpec((ROWS_BLK, D_MODEL), lambda i: (i, 0)),
        out_shape=jax.ShapeDtypeStruct((N, D_MODEL), jnp.float32),
    )(s, x2d, W, b2d)


def kernel(x, tables, W, b):
    x2d = x.reshape(N, INPUT_SIZE).astype(jnp.int32)
    tabT = tables.transpose(0, 2, 1)
    We = W[:, :D_MODEL]
    proj = _tc_project(tabT, We)
    s = _sc_embed(x2d.reshape(N * INPUT_SIZE), proj)
    out2d = _tc_cont(s, x2d, W, b.reshape(1, D_MODEL))
    return out2d.reshape(B, T, D_MODEL)


# speedup vs baseline: 2.8238x; 2.8238x over previous
import functools

import jax
import jax.numpy as jnp
from jax import lax
from jax.experimental import pallas as pl
from jax.experimental.pallas import tpu as pltpu
from jax.experimental.pallas import tpu_sc as plsc

B, T = 1024, 50
NUM_FIELDS = 26
CONT_SIZE = 13
INPUT_SIZE = NUM_FIELDS + CONT_SIZE
VOCAB = 100000
D_MODEL = 32

N = B * T
NC, NS, LANES = 2, 16, 16
NW = NC * NS
TPW = N // NW

VCHUNK = 4096
NVC = 25
VP = NVC * VCHUNK
NGRP = 6
PD = 128

CT = 16
NCHUNK = TPW // CT
RPC = CT * NUM_FIELDS
GSUB = 32
NSUB = RPC // GSUB


def _tree_sum(vals):
    while len(vals) > 1:
        nxt = [vals[k] + vals[k + 1] for k in range(0, len(vals) - 1, 2)]
        if len(vals) % 2:
            nxt.append(vals[-1])
        vals = nxt
    return vals[0]


def _proj_body(tabT_ref, w_ref, out_ref):
    lhs = tabT_ref[...].reshape(tabT_ref.shape[0] * D_MODEL, VCHUNK)
    out_ref[0] = lax.dot_general(
        lhs, w_ref[...], (((0,), (1,)), ((), ())),
        preferred_element_type=jnp.float32,
    )


def _tc_project_main(tabT, w4):
    return pl.pallas_call(
        _proj_body,
        grid=(NGRP, NVC),
        in_specs=[
            pl.BlockSpec((4, D_MODEL, VCHUNK), lambda a, c: (a, 0, c)),
            pl.BlockSpec((PD, 4 * D_MODEL), lambda a, c: (0, 0)),
        ],
        out_specs=pl.BlockSpec((1, VCHUNK, PD), lambda a, c: (a, c, 0)),
        out_shape=jax.ShapeDtypeStruct((NGRP, VP, PD), jnp.float32),
    )(tabT, w4)


def _tc_project_tail(tabT, w2):
    return pl.pallas_call(
        _proj_body,
        grid=(1, NVC),
        in_specs=[
            pl.BlockSpec((2, D_MODEL, VCHUNK), lambda a, c: (12, 0, c)),
            pl.BlockSpec((PD, 2 * D_MODEL), lambda a, c: (0, 0)),
        ],
        out_specs=pl.BlockSpec((1, VCHUNK, PD), lambda a, c: (a, c, 0)),
        out_shape=jax.ShapeDtypeStruct((1, VP, PD), jnp.float32),
    )(tabT, w2)


def _sc_embed_body(x_hbm, tab_hbm, tail_hbm, emb_hbm, xv, rid, rows, emb_v, sem):
    wid = lax.axis_index("s") * NC + lax.axis_index("c")
    base0 = wid * TPW
    iota = lax.iota(jnp.int32, LANES)

    @pl.loop(0, NCHUNK)
    def _chunk(g):
        base = base0 + g * CT
        pltpu.sync_copy(x_hbm.at[pl.ds(base * INPUT_SIZE, CT * INPUT_SIZE)], xv)

        for i in range(NUM_FIELDS):
            r = plsc.load_gather(xv, [iota * INPUT_SIZE + i])
            grp = (i // 4) if i < 24 else 0
            f = i * CT
            rid[f // GSUB, pl.ds(f % GSUB, LANES)] = r + grp * VP

        descs = [
            pltpu.async_copy(
                (tab_hbm if j < NSUB - 1 else tail_hbm).at[rid.at[j]],
                rows.at[pl.ds(j * GSUB, GSUB)],
                sem,
            )
            for j in range(NSUB)
        ]
        for d in descs:
            d.wait()

        @pl.loop(0, CT)
        def _acc(c):
            for half in range(2):
                vals = []
                for i in range(NUM_FIELDS):
                    slot = (i % 4) if i < 24 else (i - 24)
                    vals.append(
                        rows[i * CT + c,
                             pl.ds(slot * D_MODEL + half * LANES, LANES)]
                    )
                emb_v[c, pl.ds(half * LANES, LANES)] = _tree_sum(vals)

        pltpu.sync_copy(emb_v, emb_hbm.at[pl.ds(base, CT)])


_sc_embed = functools.partial(
    pl.kernel,
    out_type=jax.ShapeDtypeStruct((N, D_MODEL), jnp.float32),
    mesh=plsc.VectorSubcoreMesh(
        core_axis_name="c", subcore_axis_name="s", num_cores=NC, num_subcores=NS
    ),
    compiler_params=pltpu.CompilerParams(
        needs_layout_passes=False, use_tc_tiling_on_sc=True
    ),
    scratch_types=[
        pltpu.VMEM((CT * INPUT_SIZE,), jnp.int32),
        pltpu.VMEM((NSUB, GSUB), jnp.int32),
        pltpu.VMEM((RPC, PD), jnp.float32),
        pltpu.VMEM((CT, D_MODEL), jnp.float32),
        pltpu.SemaphoreType.DMA,
    ],
)(_sc_embed_body)


ROWS_BLK = 2048


def _tc_cont_body(s_ref, x_ref, w_ref, b_ref, out_ref):
    cont = x_ref[:, NUM_FIELDS:].astype(jnp.float32)
    out = lax.dot_general(
        cont, w_ref[:, D_MODEL:], (((1,), (1,)), ((), ())),
        preferred_element_type=jnp.float32,
    )
    out_ref[...] = out + s_ref[...] + b_ref[...]


def _tc_cont(s, x2d, W, b2d):
    return pl.pallas_call(
        _tc_cont_body,
        grid=(N // ROWS_BLK,),
        in_specs=[
            pl.BlockSpec((ROWS_BLK, D_MODEL), lambda i: (i, 0)),
            pl.BlockSpec((ROWS_BLK, INPUT_SIZE), lambda i: (i, 0)),
            pl.BlockSpec((D_MODEL, D_MODEL + CONT_SIZE), lambda i: (0, 0)),
            pl.BlockSpec((1, D_MODEL), lambda i: (0, 0)),
        ],
        out_specs=pl.BlockSpec((ROWS_BLK, D_MODEL), lambda i: (i, 0)),
        out_shape=jax.ShapeDtypeStruct((N, D_MODEL), jnp.float32),
    )(s, x2d, W, b2d)


def kernel(x, tables, W, b):
    x2d = x.reshape(N, INPUT_SIZE).astype(jnp.int32)
    tabT = tables.transpose(0, 2, 1)
    We = W[:, :D_MODEL]
    w4 = jnp.kron(jnp.eye(4, dtype=jnp.float32), We)
    w2 = jnp.pad(jnp.kron(jnp.eye(2, dtype=jnp.float32), We), ((0, 64), (0, 0)))
    proj = _tc_project_main(tabT, w4).reshape(NGRP * VP, PD)
    tail = _tc_project_tail(tabT, w2).reshape(VP, PD)
    s = _sc_embed(x2d.reshape(N * INPUT_SIZE), proj, tail)
    out2d = _tc_cont(s, x2d, W, b.reshape(1, D_MODEL))
    return out2d.reshape(B, T, D_MODEL)


# speedup vs baseline: 3.4160x; 1.2097x over previous
import functools

import jax
import jax.numpy as jnp
from jax import lax
from jax.experimental import pallas as pl
from jax.experimental.pallas import tpu as pltpu
from jax.experimental.pallas import tpu_sc as plsc

B, T = 1024, 50
NUM_FIELDS = 26
CONT_SIZE = 13
INPUT_SIZE = NUM_FIELDS + CONT_SIZE
VOCAB = 100000
D_MODEL = 32

N = B * T
NC, NS, LANES = 2, 16, 16
NW = NC * NS
TPW = N // NW

VCHUNK = 4096
NVC = 25
VP = NVC * VCHUNK
NGRP = 6
PD = 128

CT = 16
NCHUNK = TPW // CT
RPC = CT * NUM_FIELDS
GSUB = 32
NSUB = RPC // GSUB


def _tree_sum(vals):
    while len(vals) > 1:
        nxt = [vals[k] + vals[k + 1] for k in range(0, len(vals) - 1, 2)]
        if len(vals) % 2:
            nxt.append(vals[-1])
        vals = nxt
    return vals[0]


def _proj_body(tabT_ref, w_ref, out_ref):
    lhs = tabT_ref[...].reshape(tabT_ref.shape[0] * D_MODEL, VCHUNK)
    out_ref[0] = lax.dot_general(
        lhs, w_ref[...], (((0,), (1,)), ((), ())),
        preferred_element_type=jnp.float32,
    )


def _tc_project_main(tabT, w4):
    return pl.pallas_call(
        _proj_body,
        grid=(NGRP, NVC),
        in_specs=[
            pl.BlockSpec((4, D_MODEL, VCHUNK), lambda a, c: (a, 0, c)),
            pl.BlockSpec((PD, 4 * D_MODEL), lambda a, c: (0, 0)),
        ],
        out_specs=pl.BlockSpec((1, VCHUNK, PD), lambda a, c: (a, c, 0)),
        out_shape=jax.ShapeDtypeStruct((NGRP, VP, PD), jnp.float32),
    )(tabT, w4)


def _tc_project_tail(tabT, w2):
    return pl.pallas_call(
        _proj_body,
        grid=(1, NVC),
        in_specs=[
            pl.BlockSpec((2, D_MODEL, VCHUNK), lambda a, c: (12, 0, c)),
            pl.BlockSpec((PD, 2 * D_MODEL), lambda a, c: (0, 0)),
        ],
        out_specs=pl.BlockSpec((1, VCHUNK, PD), lambda a, c: (a, c, 0)),
        out_shape=jax.ShapeDtypeStruct((1, VP, PD), jnp.float32),
    )(tabT, w2)


def _sc_embed_body(
    x_hbm, tab_hbm, tail_hbm, emb_hbm,
    xv, rid_a, rid_b, rows_a, rows_b, emb_v, sem_a, sem_b,
):
    wid = lax.axis_index("s") * NC + lax.axis_index("c")
    base0 = wid * TPW
    iota = lax.iota(jnp.int32, LANES)
    rids = (rid_a, rid_b)
    rowss = (rows_a, rows_b)
    sems = (sem_a, sem_b)

    def fetch_and_fire(g, rid, rows, sem):
        base = base0 + g * CT
        pltpu.sync_copy(x_hbm.at[pl.ds(base * INPUT_SIZE, CT * INPUT_SIZE)], xv)
        for i in range(NUM_FIELDS):
            r = plsc.load_gather(xv, [iota * INPUT_SIZE + i])
            grp = (i // 4) if i < 24 else 0
            f = i * CT
            rid[f // GSUB, pl.ds(f % GSUB, LANES)] = r + grp * VP
        for j in range(NSUB):
            pltpu.async_copy(
                (tab_hbm if j < NSUB - 1 else tail_hbm).at[rid.at[j]],
                rows.at[pl.ds(j * GSUB, GSUB)],
                sem,
            )

    def drain(rid, rows, sem):
        for j in range(NSUB):
            pltpu.make_async_copy(
                (tab_hbm if j < NSUB - 1 else tail_hbm).at[rid.at[j]],
                rows.at[pl.ds(j * GSUB, GSUB)],
                sem,
            ).wait()

    def accumulate(g, rows):
        @pl.loop(0, CT)
        def _acc(c):
            for half in range(2):
                vals = []
                for i in range(NUM_FIELDS):
                    slot = (i % 4) if i < 24 else (i - 24)
                    vals.append(
                        rows[i * CT + c,
                             pl.ds(slot * D_MODEL + half * LANES, LANES)]
                    )
                emb_v[c, pl.ds(half * LANES, LANES)] = _tree_sum(vals)

        pltpu.sync_copy(emb_v, emb_hbm.at[pl.ds(base0 + g * CT, CT)])

    fetch_and_fire(0, rid_a, rows_a, sem_a)

    @pl.loop(0, NCHUNK // 2)
    def _pair(h):
        for p in range(2):
            g = 2 * h + p
            nxt = jnp.minimum(g + 1, NCHUNK - 1)
            fetch_and_fire(nxt, rids[1 - p], rowss[1 - p], sems[1 - p])
            drain(rids[p], rowss[p], sems[p])
            accumulate(g, rowss[p])

    drain(rid_a, rows_a, sem_a)


_sc_embed = functools.partial(
    pl.kernel,
    out_type=jax.ShapeDtypeStruct((N, D_MODEL), jnp.float32),
    mesh=plsc.VectorSubcoreMesh(
        core_axis_name="c", subcore_axis_name="s", num_cores=NC, num_subcores=NS
    ),
    compiler_params=pltpu.CompilerParams(
        needs_layout_passes=False, use_tc_tiling_on_sc=True
    ),
    scratch_types=[
        pltpu.VMEM((CT * INPUT_SIZE,), jnp.int32),
        pltpu.VMEM((NSUB, GSUB), jnp.int32),
        pltpu.VMEM((NSUB, GSUB), jnp.int32),
        pltpu.VMEM((RPC, PD), jnp.float32),
        pltpu.VMEM((RPC, PD), jnp.float32),
        pltpu.VMEM((CT, D_MODEL), jnp.float32),
        pltpu.SemaphoreType.DMA,
        pltpu.SemaphoreType.DMA,
    ],
)(_sc_embed_body)


ROWS_BLK = 2048


def _tc_cont_body(s_ref, x_ref, w_ref, b_ref, out_ref):
    cont = x_ref[:, NUM_FIELDS:].astype(jnp.float32)
    out = lax.dot_general(
        cont, w_ref[:, D_MODEL:], (((1,), (1,)), ((), ())),
        preferred_element_type=jnp.float32,
    )
    out_ref[...] = out + s_ref[...] + b_ref[...]


def _tc_cont(s, x2d, W, b2d):
    return pl.pallas_call(
        _tc_cont_body,
        grid=(N // ROWS_BLK,),
        in_specs=[
            pl.BlockSpec((ROWS_BLK, D_MODEL), lambda i: (i, 0)),
            pl.BlockSpec((ROWS_BLK, INPUT_SIZE), lambda i: (i, 0)),
            pl.BlockSpec((D_MODEL, D_MODEL + CONT_SIZE), lambda i: (0, 0)),
            pl.BlockSpec((1, D_MODEL), lambda i: (0, 0)),
        ],
        out_specs=pl.BlockSpec((ROWS_BLK, D_MODEL), lambda i: (i, 0)),
        out_shape=jax.ShapeDtypeStruct((N, D_MODEL), jnp.float32),
    )(s, x2d, W, b2d)


def kernel(x, tables, W, b):
    x2d = x.reshape(N, INPUT_SIZE).astype(jnp.int32)
    tabT = tables.transpose(0, 2, 1)
    We = W[:, :D_MODEL]
    w4 = jnp.kron(jnp.eye(4, dtype=jnp.float32), We)
    w2 = jnp.pad(jnp.kron(jnp.eye(2, dtype=jnp.float32), We), ((0, 64), (0, 0)))
    proj = _tc_project_main(tabT, w4).reshape(NGRP * VP, PD)
    tail = _tc_project_tail(tabT, w2).reshape(VP, PD)
    s = _sc_embed(x2d.reshape(N * INPUT_SIZE), proj, tail)
    out2d = _tc_cont(s, x2d, W, b.reshape(1, D_MODEL))
    return out2d.reshape(B, T, D_MODEL)


# speedup vs baseline: 3.4839x; 1.0199x over previous
import functools

import jax
import jax.numpy as jnp
from jax import lax
from jax.experimental import pallas as pl
from jax.experimental.pallas import tpu as pltpu
from jax.experimental.pallas import tpu_sc as plsc

B, T = 1024, 50
NUM_FIELDS = 26
CONT_SIZE = 13
INPUT_SIZE = NUM_FIELDS + CONT_SIZE
VOCAB = 100000
D_MODEL = 32

N = B * T
NC, NS, LANES = 2, 16, 16
NW = NC * NS
TPW = N // NW

VCHUNK = 4096
NVC = 25
VP = NVC * VCHUNK
NGRP = 6
PD = 128

CT = 16
NCHUNK = TPW // CT
RPC = CT * NUM_FIELDS
GSUB = 32
NSUB = RPC // GSUB


def _tree_sum(vals):
    while len(vals) > 1:
        nxt = [vals[k] + vals[k + 1] for k in range(0, len(vals) - 1, 2)]
        if len(vals) % 2:
            nxt.append(vals[-1])
        vals = nxt
    return vals[0]


def _proj_body(tabT_ref, w_ref, out_ref):
    lhs = tabT_ref[...].reshape(tabT_ref.shape[0] * D_MODEL, VCHUNK)
    out_ref[0] = lax.dot_general(
        lhs, w_ref[...], (((0,), (1,)), ((), ())),
        preferred_element_type=jnp.float32,
    )


def _tc_project_main(tabT, w4):
    return pl.pallas_call(
        _proj_body,
        grid=(NGRP, NVC),
        in_specs=[
            pl.BlockSpec((4, D_MODEL, VCHUNK), lambda a, c: (a, 0, c)),
            pl.BlockSpec((PD, 4 * D_MODEL), lambda a, c: (0, 0)),
        ],
        out_specs=pl.BlockSpec((1, VCHUNK, PD), lambda a, c: (a, c, 0)),
        out_shape=jax.ShapeDtypeStruct((NGRP, VP, PD), jnp.float32),
    )(tabT, w4)


def _tc_project_tail(tabT, w2):
    return pl.pallas_call(
        _proj_body,
        grid=(1, NVC),
        in_specs=[
            pl.BlockSpec((2, D_MODEL, VCHUNK), lambda a, c: (12, 0, c)),
            pl.BlockSpec((PD, 2 * D_MODEL), lambda a, c: (0, 0)),
        ],
        out_specs=pl.BlockSpec((1, VCHUNK, PD), lambda a, c: (a, c, 0)),
        out_shape=jax.ShapeDtypeStruct((1, VP, PD), jnp.float32),
    )(tabT, w2)


def _sc_embed_body(
    x_hbm, tab_hbm, tail_hbm, emb_hbm,
    xv, rid_a, rid_b, rows_a, rows_b, emb_v, sem_a, sem_b,
):
    wid = lax.axis_index("s") * NC + lax.axis_index("c")
    base0 = wid * TPW
    iota = lax.iota(jnp.int32, LANES)
    rids = (rid_a, rid_b)
    rowss = (rows_a, rows_b)
    sems = (sem_a, sem_b)

    def fetch_and_fire(g, rid, rows, sem):
        base = base0 + g * CT
        pltpu.sync_copy(x_hbm.at[pl.ds(base, CT)], xv)
        for i in range(NUM_FIELDS):
            r = plsc.load_gather(xv, [iota, jnp.full((LANES,), i, jnp.int32)])
            grp = (i // 4) if i < 24 else 0
            f = i * CT
            rid[f // GSUB, pl.ds(f % GSUB, LANES)] = r + grp * VP
        for j in range(NSUB):
            pltpu.async_copy(
                (tab_hbm if j < NSUB - 1 else tail_hbm).at[rid.at[j]],
                rows.at[pl.ds(j * GSUB, GSUB)],
                sem,
            )

    def drain(rid, rows, sem):
        for j in range(NSUB):
            pltpu.make_async_copy(
                (tab_hbm if j < NSUB - 1 else tail_hbm).at[rid.at[j]],
                rows.at[pl.ds(j * GSUB, GSUB)],
                sem,
            ).wait()

    def accumulate(g, rows):
        @pl.loop(0, CT)
        def _acc(c):
            for half in range(2):
                vals = []
                for i in range(NUM_FIELDS):
                    slot = (i % 4) if i < 24 else (i - 24)
                    vals.append(
                        rows[i * CT + c,
                             pl.ds(slot * D_MODEL + half * LANES, LANES)]
                    )
                emb_v[c, pl.ds(half * LANES, LANES)] = _tree_sum(vals)

        pltpu.sync_copy(emb_v, emb_hbm.at[pl.ds(base0 + g * CT, CT)])

    fetch_and_fire(0, rid_a, rows_a, sem_a)

    @pl.loop(0, NCHUNK // 2)
    def _pair(h):
        for p in range(2):
            g = 2 * h + p
            nxt = jnp.minimum(g + 1, NCHUNK - 1)
            fetch_and_fire(nxt, rids[1 - p], rowss[1 - p], sems[1 - p])
            drain(rids[p], rowss[p], sems[p])
            accumulate(g, rowss[p])

    drain(rid_a, rows_a, sem_a)


_sc_embed = functools.partial(
    pl.kernel,
    out_type=jax.ShapeDtypeStruct((N, D_MODEL), jnp.float32),
    mesh=plsc.VectorSubcoreMesh(
        core_axis_name="c", subcore_axis_name="s", num_cores=NC, num_subcores=NS
    ),
    compiler_params=pltpu.CompilerParams(
        needs_layout_passes=False, use_tc_tiling_on_sc=True
    ),
    scratch_types=[
        pltpu.VMEM((CT, INPUT_SIZE), jnp.int32),
        pltpu.VMEM((NSUB, GSUB), jnp.int32),
        pltpu.VMEM((NSUB, GSUB), jnp.int32),
        pltpu.VMEM((RPC, PD), jnp.float32),
        pltpu.VMEM((RPC, PD), jnp.float32),
        pltpu.VMEM((CT, D_MODEL), jnp.float32),
        pltpu.SemaphoreType.DMA,
        pltpu.SemaphoreType.DMA,
    ],
)(_sc_embed_body)


ROWS_BLK = 2048


def _tc_cont_body(s_ref, x_ref, w_ref, b_ref, out_ref):
    cont = x_ref[:, NUM_FIELDS:].astype(jnp.float32)
    out = lax.dot_general(
        cont, w_ref[:, D_MODEL:], (((1,), (1,)), ((), ())),
        preferred_element_type=jnp.float32,
    )
    out_ref[...] = out + s_ref[...] + b_ref[...]


def _tc_cont(s, x2d, W, b2d):
    return pl.pallas_call(
        _tc_cont_body,
        grid=(N // ROWS_BLK,),
        in_specs=[
            pl.BlockSpec((ROWS_BLK, D_MODEL), lambda i: (i, 0)),
            pl.BlockSpec((ROWS_BLK, INPUT_SIZE), lambda i: (i, 0)),
            pl.BlockSpec((D_MODEL, D_MODEL + CONT_SIZE), lambda i: (0, 0)),
            pl.BlockSpec((1, D_MODEL), lambda i: (0, 0)),
        ],
        out_specs=pl.BlockSpec((ROWS_BLK, D_MODEL), lambda i: (i, 0)),
        out_shape=jax.ShapeDtypeStruct((N, D_MODEL), jnp.float32),
    )(s, x2d, W, b2d)


def kernel(x, tables, W, b):
    x2d = x.reshape(N, INPUT_SIZE).astype(jnp.int32)
    tabT = tables.transpose(0, 2, 1)
    We = W[:, :D_MODEL]
    w4 = jnp.kron(jnp.eye(4, dtype=jnp.float32), We)
    w2 = jnp.pad(jnp.kron(jnp.eye(2, dtype=jnp.float32), We), ((0, 64), (0, 0)))
    proj = _tc_project_main(tabT, w4).reshape(NGRP * VP, PD)
    tail = _tc_project_tail(tabT, w2).reshape(VP, PD)
    s = _sc_embed(x2d, proj, tail)
    out2d = _tc_cont(s, x2d, W, b.reshape(1, D_MODEL))
    return out2d.reshape(B, T, D_MODEL)
